# baseline (speedup 1.0000x reference)
import functools

import jax
import jax.numpy as jnp
from jax import lax
from jax.experimental import pallas as pl
from jax.experimental.pallas import tpu as pltpu
from jax.experimental.pallas import tpu_sc as plsc

B, H, W = 4, 512, 512
KH, KW = 32, 32
K = KH * KW
YX_SCALE = KH / (0.4 * H)
LAB_SCALE = 0.26
NUM_UPDATES = 4

NC, NS, L = 2, 16, 16
ROWS_PER_TILE = H // NS
VECS_PER_ROW = W // L
ST = 6
ACC_W = K * ST
SEG_PER_TILE = K // NS
CH_W = SEG_PER_TILE * ST


def _ssn_body(img_ref, sidx_ref, pfeat_ref, spfeat_ref, assoc_ref, fidx_ref,
              acc, table, red, labbuf, idxbuf, tstage, spout,
              assocbuf, fidxbuf, pfeatbuf, slots, shtab, sem_in, sem_in2,
              sem_out):
  core = lax.axis_index("c")
  tile = lax.axis_index("s")
  iota = lax.iota(jnp.int32, 16)
  lane_base = iota * ACC_W
  zeros16 = jnp.zeros((16,), jnp.float32)
  ones16 = jnp.ones((16,), jnp.float32)

  def _tree_sum(vals):
    while len(vals) > 1:
      vals = [a + b for a, b in zip(vals[::2], vals[1::2])]
    return vals[0]

  @plsc.parallel_loop(0, (L * ACC_W) // (16 * 8), unroll=4)
  def _zero_body(i):
    for u in range(8):
      acc[pl.ds((i * 8 + u) * 16, 16)] = zeros16

  def _fire_row(b, r, par, sem):
    pltpu.async_copy(sidx_ref.at[b, 0, r], idxbuf.at[pl.ds(par * W, W)], sem)
    for c in range(3):
      pltpu.async_copy(img_ref.at[b, c, r],
                       labbuf.at[pl.ds((par * 3 + c) * W, W)], sem)

  def _drain_row(b, r, par, sem):
    pltpu.make_async_copy(sidx_ref.at[b, 0, r],
                          idxbuf.at[pl.ds(par * W, W)], sem).wait()
    for c in range(3):
      pltpu.make_async_copy(img_ref.at[b, c, r],
                            labbuf.at[pl.ds((par * 3 + c) * W, W)], sem).wait()

  def run_rows(b, vec_loop):
    _fire_row(b, tile * ROWS_PER_TILE, 0, sem_in)
    def row_body(ri, carry):
      r = tile * ROWS_PER_TILE + ri
      even = (ri & 1) == 0
      last = ri >= ROWS_PER_TILE - 1

      @pl.when(jnp.logical_and(jnp.logical_not(last), even))
      def _():
        _fire_row(b, r + 1, 1, sem_in2)

      @pl.when(jnp.logical_and(jnp.logical_not(last), jnp.logical_not(even)))
      def _():
        _fire_row(b, r + 1, 0, sem_in)

      @pl.when(even)
      def _():
        _drain_row(b, r, 0, sem_in)

      @pl.when(jnp.logical_not(even))
      def _():
        _drain_row(b, r, 1, sem_in2)

      vec_loop(ri, r, ri & 1)
      return carry
    lax.fori_loop(0, ROWS_PER_TILE, row_body, 0)

  def pixel_feats(r, v, par):
    y16 = lax.broadcast_in_dim(r, (16,), ())
    p0 = y16.astype(jnp.float32) * YX_SCALE
    x16 = iota + v * 16
    p1 = x16.astype(jnp.float32) * YX_SCALE
    lab = [labbuf[pl.ds((par * 3 + c) * W + v * 16, 16)] * LAB_SCALE
           for c in range(3)]
    return [p0, p1] + lab

  def cand_k6(s_i):
    sh = s_i >> 5
    sw = s_i & 31
    shm = jnp.maximum(sh - 1, 0)
    shp = jnp.minimum(sh + 1, KH - 1)
    swm = jnp.maximum(sw - 1, 0)
    swp = jnp.minimum(sw + 1, KW - 1)
    rh6 = [shm * (KW * ST), sh * (KW * ST), shp * (KW * ST)]
    cw6 = [swm * ST, sw * ST, swp * ST]
    return [rh6[a] + cw6[bb] for a in range(3) for bb in range(3)]

  def softmax9_dot(p, k6s):
    lams = []
    m = None
    for k6 in k6s:
      lam = p[0] * plsc.load_gather(table, [k6])
      for c in range(1, 5):
        lam = lam + p[c] * plsc.load_gather(table, [k6 + c])
      lam = lam - plsc.load_gather(table, [k6 + 5])
      lams.append(lam)
      m = lam if m is None else jnp.maximum(m, lam)
    es = [jnp.exp(lam - m) for lam in lams]
    ssum = None
    for e in es:
      ssum = e if ssum is None else ssum + e
    inv = 1.0 / ssum
    return es, inv

  def softmax9(p, k6s):
    lams = []
    m = None
    for k6 in k6s:
      d = None
      for c in range(5):
        fc = plsc.load_gather(table, [k6 + c if c else k6])
        df = p[c] - fc
        sq = df * df
        d = sq if d is None else d + sq
      lam = -d
      lams.append(lam)
      m = lam if m is None else jnp.maximum(m, lam)
    es = [jnp.exp(lam - m) for lam in lams]
    ssum = None
    for e in es:
      ssum = e if ssum is None else ssum + e
    inv = 1.0 / ssum
    return es, inv

  def phase_init(b):
    def vec_loop(ri, r, par):
      @plsc.parallel_loop(0, VECS_PER_ROW, unroll=2)
      def vec_body(v):
        s_i = idxbuf[pl.ds(par * W + v * 16, 16)]
        p = pixel_feats(r, v, par)
        base = s_i * ST + lane_base
        for c in range(5):
          plsc.addupdate_scatter(acc, [base + c if c else base], p[c])
        plsc.addupdate_scatter(acc, [base + 5], ones16)
    run_rows(b, vec_loop)

  def phase_update(b):
    def vec_loop(ri, r, par):
      def vec_body(v, vc):
        s_i = idxbuf[pl.ds(par * W + v * 16, 16)]
        p = pixel_feats(r, v, par)
        k6s = cand_k6(s_i)
        es, inv = softmax9_dot(p, k6s)
        for j in range(9):
          a = es[j] * inv
          base = k6s[j] + lane_base
          for c in range(5):
            plsc.addupdate_scatter(acc, [base + c if c else base], a * p[c])
          plsc.addupdate_scatter(acc, [base + 5], a)
        return vc
      lax.fori_loop(0, VECS_PER_ROW, vec_body, 0)
    run_rows(b, vec_loop)

  def phase_reduce(b, is_init, last_f):
    @plsc.parallel_loop(0, ACC_W // 16, unroll=4)
    def red_body(i):
      base = i * 16
      vals = [acc[pl.ds(l * ACC_W + base, 16)] for l in range(L)]
      red[pl.ds(base, 16)] = _tree_sum(vals)
      for l in range(L):
        acc[pl.ds(l * ACC_W + base, 16)] = zeros16
    pltpu.sync_copy(red, slots.at[pl.ds(tile * ACC_W, ACC_W)])
    plsc.subcore_barrier()
    cps = [pltpu.async_copy(slots.at[pl.ds(i * ACC_W + tile * CH_W, CH_W)],
                            red.at[pl.ds(i * CH_W, CH_W)], sem_in)
           for i in range(NS)]
    for cp in cps:
      cp.wait()
    @plsc.parallel_loop(0, CH_W // 16, unroll=2)
    def sum_body(i):
      base = i * 16
      vals = [red[pl.ds(l * CH_W + base, 16)] for l in range(NS)]
      tstage[pl.ds(base, 16)] = _tree_sum(vals)
    for vv in range(SEG_PER_TILE // 16):
      kloc6 = (iota + vv * 16) * ST
      num = [plsc.load_gather(tstage, [kloc6 + c if c else kloc6])
             for c in range(5)]
      den = plsc.load_gather(tstage, [kloc6 + 5])
      if is_init:
        dadj = jnp.maximum(den, 1e-12)
      else:
        dadj = den + 1e-10
      invd = 1.0 / dadj
      fcs = [num[c] * invd for c in range(5)]
      sqs = [fc * fc for fc in fcs]
      nrm = ((sqs[0] + sqs[1]) + (sqs[2] + sqs[3])) + sqs[4]
      for c in range(5):
        val = jnp.where(last_f, fcs[c], fcs[c] * 2.0)
        plsc.store_scatter(tstage, [kloc6 + c if c else kloc6], val)
        if not is_init:
          spout[pl.ds(c * SEG_PER_TILE + vv * 16, 16)] = fcs[c]
      plsc.store_scatter(tstage, [kloc6 + 5], nrm)
    pltpu.sync_copy(tstage, shtab.at[pl.ds(tile * CH_W, CH_W)])
    if not is_init:
      for c in range(5):
        pltpu.sync_copy(spout.at[pl.ds(c * SEG_PER_TILE, SEG_PER_TILE)],
                        spfeat_ref.at[b, c, pl.ds(tile * SEG_PER_TILE,
                                                  SEG_PER_TILE)])
    plsc.subcore_barrier()
    pltpu.sync_copy(shtab, table)

  def phase_final(b):
    def vec_loop(ri, r, par):
      @pl.when(ri > 0)
      def _drain():
        _fire_outputs(b, r - 1, wait_only=True)

      def vec_body(v, vc):
        s_i = idxbuf[pl.ds(par * W + v * 16, 16)]
        p = pixel_feats(r, v, par)
        sh = s_i >> 5
        sw = s_i & 31
        shm = jnp.maximum(sh - 1, 0)
        shp = jnp.minimum(sh + 1, KH - 1)
        swm = jnp.maximum(sw - 1, 0)
        swp = jnp.minimum(sw + 1, KW - 1)
        rh = [shm * KW, sh * KW, shp * KW]
        cw = [swm, sw, swp]
        ks = [rh[a] + cw[bb] for a in range(3) for bb in range(3)]
        k6s = [kk * ST for kk in ks]
        es, inv = softmax9(p, k6s)
        best = None
        bk = None
        for j in range(9):
          a = es[j] * inv
          assocbuf[pl.ds(j * W + v * 16, 16)] = a
          if j == 0:
            best, bk = a, ks[0]
          else:
            gt = a > best
            best = jnp.where(gt, a, best)
            bk = jnp.where(gt, ks[j], bk)
        fidxbuf[pl.ds(v * 16, 16)] = bk
        for c in range(5):
          pfeatbuf[pl.ds(c * W + v * 16, 16)] = p[c]
        return vc
      lax.fori_loop(0, VECS_PER_ROW, vec_body, 0)
      _fire_outputs(b, r, wait_only=False)
    run_rows(b, vec_loop)
    _fire_outputs(b, tile * ROWS_PER_TILE + ROWS_PER_TILE - 1, wait_only=True)

  def _fire_outputs(b, r, wait_only):
    descs = []
    for j in range(9):
      descs.append((assocbuf.at[pl.ds(j * W, W)], assoc_ref.at[b, j, r]))
    descs.append((fidxbuf, fidx_ref.at[b, 0, r]))
    for c in range(5):
      descs.append((pfeatbuf.at[pl.ds(c * W, W)], pfeat_ref.at[b, c, r]))
    if wait_only:
      for src, dst in descs:
        pltpu.make_async_copy(src, dst, sem_out).wait()
    else:
      for src, dst in descs:
        pltpu.async_copy(src, dst, sem_out)

  for bl in range(B // NC):
    b = core * (B // NC) + bl
    phase_init(b)
    phase_reduce(b, is_init=True, last_f=False)
    def upd_body(it, carry):
      phase_update(b)
      phase_reduce(b, is_init=False, last_f=(it == NUM_UPDATES - 1))
      return carry
    lax.fori_loop(0, NUM_UPDATES, upd_body, 0)
    phase_final(b)


@jax.jit
def _ssn(img_lab, init_spIndx):
  mesh = plsc.VectorSubcoreMesh(core_axis_name="c", subcore_axis_name="s")
  f = pl.kernel(
      _ssn_body,
      out_type=[
          jax.ShapeDtypeStruct((B, 5, H, W), jnp.float32),
          jax.ShapeDtypeStruct((B, 5, K), jnp.float32),
          jax.ShapeDtypeStruct((B, 9, H, W), jnp.float32),
          jax.ShapeDtypeStruct((B, 1, H, W), jnp.int32),
      ],
      mesh=mesh,
      compiler_params=pltpu.CompilerParams(needs_layout_passes=False),
      scratch_types=[
          pltpu.VMEM((L * ACC_W,), jnp.float32),
          pltpu.VMEM((ACC_W,), jnp.float32),
          pltpu.VMEM((ACC_W,), jnp.float32),
          pltpu.VMEM((2 * 3 * W,), jnp.float32),
          pltpu.VMEM((2 * W,), jnp.int32),
          pltpu.VMEM((CH_W,), jnp.float32),
          pltpu.VMEM((5 * SEG_PER_TILE,), jnp.float32),
          pltpu.VMEM((9 * W,), jnp.float32),
          pltpu.VMEM((W,), jnp.int32),
          pltpu.VMEM((5 * W,), jnp.float32),
          pltpu.VMEM_SHARED((NS * ACC_W,), jnp.float32),
          pltpu.VMEM_SHARED((ACC_W,), jnp.float32),
          pltpu.SemaphoreType.DMA,
          pltpu.SemaphoreType.DMA,
          pltpu.SemaphoreType.DMA,
      ],
  )
  return f(img_lab, init_spIndx.astype(jnp.int32))


def kernel(img_lab, init_spIndx):
  pFeat, spFeat, psp_assoc, final_spIndx = _ssn(img_lab, init_spIndx)
  return (pFeat, spFeat, psp_assoc, final_spIndx)


# speedup vs baseline: 1.0069x; 1.0069x over previous
import functools

import jax
import jax.numpy as jnp
from jax import lax
from jax.experimental import pallas as pl
from jax.experimental.pallas import tpu as pltpu
from jax.experimental.pallas import tpu_sc as plsc

B, H, W = 4, 512, 512
KH, KW = 32, 32
K = KH * KW
YX_SCALE = KH / (0.4 * H)
LAB_SCALE = 0.26
NUM_UPDATES = 4

NC, NS, L = 2, 16, 16
ROWS_PER_TILE = H // NS
VECS_PER_ROW = W // L
ST = 6
ACC_W = K * ST
SEG_PER_TILE = K // NS
CH_W = SEG_PER_TILE * ST


def _ssn_body(img_ref, sidx_ref, pfeat_ref, spfeat_ref, assoc_ref, fidx_ref,
              acc, table, red, labbuf, idxbuf, tstage, spout,
              assocbuf, fidxbuf, pfeatbuf, slots, shtab, sem_in, sem_in2,
              sem_out):
  core = lax.axis_index("c")
  tile = lax.axis_index("s")
  iota = lax.iota(jnp.int32, 16)
  lane_base = iota * ACC_W
  zeros16 = jnp.zeros((16,), jnp.float32)
  ones16 = jnp.ones((16,), jnp.float32)

  def _tree_sum(vals):
    while len(vals) > 1:
      vals = [a + b for a, b in zip(vals[::2], vals[1::2])]
    return vals[0]

  @plsc.parallel_loop(0, (L * ACC_W) // (16 * 8), unroll=4)
  def _zero_body(i):
    for u in range(8):
      acc[pl.ds((i * 8 + u) * 16, 16)] = zeros16

  def _fire_row(b, r, par, sem):
    pltpu.async_copy(sidx_ref.at[b, 0, r], idxbuf.at[pl.ds(par * W, W)], sem)
    for c in range(3):
      pltpu.async_copy(img_ref.at[b, c, r],
                       labbuf.at[pl.ds((par * 3 + c) * W, W)], sem)

  def _drain_row(b, r, par, sem):
    pltpu.make_async_copy(sidx_ref.at[b, 0, r],
                          idxbuf.at[pl.ds(par * W, W)], sem).wait()
    for c in range(3):
      pltpu.make_async_copy(img_ref.at[b, c, r],
                            labbuf.at[pl.ds((par * 3 + c) * W, W)], sem).wait()

  def run_rows(b, vec_loop):
    _fire_row(b, tile * ROWS_PER_TILE, 0, sem_in)
    def row_body(ri, carry):
      r = tile * ROWS_PER_TILE + ri
      even = (ri & 1) == 0
      last = ri >= ROWS_PER_TILE - 1

      @pl.when(jnp.logical_and(jnp.logical_not(last), even))
      def _():
        _fire_row(b, r + 1, 1, sem_in2)

      @pl.when(jnp.logical_and(jnp.logical_not(last), jnp.logical_not(even)))
      def _():
        _fire_row(b, r + 1, 0, sem_in)

      @pl.when(even)
      def _():
        _drain_row(b, r, 0, sem_in)

      @pl.when(jnp.logical_not(even))
      def _():
        _drain_row(b, r, 1, sem_in2)

      vec_loop(ri, r, ri & 1)
      return carry
    lax.fori_loop(0, ROWS_PER_TILE, row_body, 0)

  def pixel_feats(r, v, par):
    y16 = lax.broadcast_in_dim(r, (16,), ())
    p0 = y16.astype(jnp.float32) * YX_SCALE
    x16 = iota + v * 16
    p1 = x16.astype(jnp.float32) * YX_SCALE
    lab = [labbuf[pl.ds((par * 3 + c) * W + v * 16, 16)] * LAB_SCALE
           for c in range(3)]
    return [p0, p1] + lab

  def cand_parts(s_i):
    sh = s_i >> 5
    sw = s_i & 31
    shm = jnp.maximum(sh - 1, 0)
    shp = jnp.minimum(sh + 1, KH - 1)
    swm = jnp.maximum(sw - 1, 0)
    swp = jnp.minimum(sw + 1, KW - 1)
    rh6 = [shm * (KW * ST), sh * (KW * ST), shp * (KW * ST)]
    cw6 = [swm * ST, sw * ST, swp * ST]
    return rh6, cw6

  def softmax9_dot(p, k6s):
    lams = []
    m = None
    for k6 in k6s:
      lam = p[0] * plsc.load_gather(table, [k6])
      for c in range(1, 5):
        lam = lam + p[c] * plsc.load_gather(table, [k6 + c])
      lam = lam - plsc.load_gather(table, [k6 + 5])
      lams.append(lam)
      m = lam if m is None else jnp.maximum(m, lam)
    es = [jnp.exp(lam - m) for lam in lams]
    ssum = None
    for e in es:
      ssum = e if ssum is None else ssum + e
    inv = 1.0 / ssum
    return es, inv

  def softmax9(p, k6s):
    lams = []
    m = None
    for k6 in k6s:
      d = None
      for c in range(5):
        fc = plsc.load_gather(table, [k6 + c if c else k6])
        df = p[c] - fc
        sq = df * df
        d = sq if d is None else d + sq
      lam = -d
      lams.append(lam)
      m = lam if m is None else jnp.maximum(m, lam)
    es = [jnp.exp(lam - m) for lam in lams]
    ssum = None
    for e in es:
      ssum = e if ssum is None else ssum + e
    inv = 1.0 / ssum
    return es, inv

  def phase_init(b):
    def vec_loop(ri, r, par):
      @plsc.parallel_loop(0, VECS_PER_ROW, unroll=2)
      def vec_body(v):
        s_i = idxbuf[pl.ds(par * W + v * 16, 16)]
        p = pixel_feats(r, v, par)
        base = s_i * ST + lane_base
        for c in range(5):
          plsc.addupdate_scatter(acc, [base + c if c else base], p[c])
        plsc.addupdate_scatter(acc, [base + 5], ones16)
    run_rows(b, vec_loop)

  def phase_update(b):
    def vec_loop(ri, r, par):
      def vec_body(v, vc):
        s_i = idxbuf[pl.ds(par * W + v * 16, 16)]
        p = pixel_feats(r, v, par)
        rh6, cw6 = cand_parts(s_i)
        k6s = [rh6[a] + cw6[bb] for a in range(3) for bb in range(3)]
        es, inv = softmax9_dot(p, k6s)
        for j in range(9):
          a = es[j] * inv
          base = (rh6[j // 3] + cw6[j % 3]) + lane_base
          for c in range(5):
            plsc.addupdate_scatter(acc, [base + c if c else base], a * p[c])
          plsc.addupdate_scatter(acc, [base + 5], a)
        return vc
      lax.fori_loop(0, VECS_PER_ROW, vec_body, 0)
    run_rows(b, vec_loop)

  def phase_reduce(b, is_init, last_f):
    @plsc.parallel_loop(0, ACC_W // 16, unroll=2)
    def red_body(i):
      base = i * 16
      vals = [acc[pl.ds(l * ACC_W + base, 16)] for l in range(L)]
      red[pl.ds(base, 16)] = _tree_sum(vals)
      for l in range(L):
        acc[pl.ds(l * ACC_W + base, 16)] = zeros16
    pltpu.sync_copy(red, slots.at[pl.ds(tile * ACC_W, ACC_W)])
    plsc.subcore_barrier()
    cps = [pltpu.async_copy(slots.at[pl.ds(i * ACC_W + tile * CH_W, CH_W)],
                            red.at[pl.ds(i * CH_W, CH_W)], sem_in)
           for i in range(NS)]
    for cp in cps:
      cp.wait()
    @plsc.parallel_loop(0, CH_W // 16, unroll=2)
    def sum_body(i):
      base = i * 16
      vals = [red[pl.ds(l * CH_W + base, 16)] for l in range(NS)]
      tstage[pl.ds(base, 16)] = _tree_sum(vals)
    for vv in range(SEG_PER_TILE // 16):
      kloc6 = (iota + vv * 16) * ST
      num = [plsc.load_gather(tstage, [kloc6 + c if c else kloc6])
             for c in range(5)]
      den = plsc.load_gather(tstage, [kloc6 + 5])
      if is_init:
        dadj = jnp.maximum(den, 1e-12)
      else:
        dadj = den + 1e-10
      invd = 1.0 / dadj
      fcs = [num[c] * invd for c in range(5)]
      sqs = [fc * fc for fc in fcs]
      nrm = ((sqs[0] + sqs[1]) + (sqs[2] + sqs[3])) + sqs[4]
      for c in range(5):
        val = jnp.where(last_f, fcs[c], fcs[c] * 2.0)
        plsc.store_scatter(tstage, [kloc6 + c if c else kloc6], val)
        if not is_init:
          spout[pl.ds(c * SEG_PER_TILE + vv * 16, 16)] = fcs[c]
      plsc.store_scatter(tstage, [kloc6 + 5], nrm)
    pltpu.sync_copy(tstage, shtab.at[pl.ds(tile * CH_W, CH_W)])
    if not is_init:
      for c in range(5):
        pltpu.sync_copy(spout.at[pl.ds(c * SEG_PER_TILE, SEG_PER_TILE)],
                        spfeat_ref.at[b, c, pl.ds(tile * SEG_PER_TILE,
                                                  SEG_PER_TILE)])
    plsc.subcore_barrier()
    pltpu.sync_copy(shtab, table)

  def phase_final(b):
    def vec_loop(ri, r, par):
      @pl.when(ri > 0)
      def _drain():
        _fire_outputs(b, r - 1, wait_only=True)

      def vec_body(v, vc):
        s_i = idxbuf[pl.ds(par * W + v * 16, 16)]
        p = pixel_feats(r, v, par)
        sh = s_i >> 5
        sw = s_i & 31
        shm = jnp.maximum(sh - 1, 0)
        shp = jnp.minimum(sh + 1, KH - 1)
        swm = jnp.maximum(sw - 1, 0)
        swp = jnp.minimum(sw + 1, KW - 1)
        rh = [shm * KW, sh * KW, shp * KW]
        cw = [swm, sw, swp]
        ks = [rh[a] + cw[bb] for a in range(3) for bb in range(3)]
        k6s = [kk * ST for kk in ks]
        es, inv = softmax9(p, k6s)
        best = None
        bk = None
        for j in range(9):
          a = es[j] * inv
          assocbuf[pl.ds(j * W + v * 16, 16)] = a
          if j == 0:
            best, bk = a, ks[0]
          else:
            gt = a > best
            best = jnp.where(gt, a, best)
            bk = jnp.where(gt, ks[j], bk)
        fidxbuf[pl.ds(v * 16, 16)] = bk
        for c in range(5):
          pfeatbuf[pl.ds(c * W + v * 16, 16)] = p[c]
        return vc
      lax.fori_loop(0, VECS_PER_ROW, vec_body, 0)
      _fire_outputs(b, r, wait_only=False)
    run_rows(b, vec_loop)
    _fire_outputs(b, tile * ROWS_PER_TILE + ROWS_PER_TILE - 1, wait_only=True)

  def _fire_outputs(b, r, wait_only):
    descs = []
    for j in range(9):
      descs.append((assocbuf.at[pl.ds(j * W, W)], assoc_ref.at[b, j, r]))
    descs.append((fidxbuf, fidx_ref.at[b, 0, r]))
    for c in range(5):
      descs.append((pfeatbuf.at[pl.ds(c * W, W)], pfeat_ref.at[b, c, r]))
    if wait_only:
      for src, dst in descs:
        pltpu.make_async_copy(src, dst, sem_out).wait()
    else:
      for src, dst in descs:
        pltpu.async_copy(src, dst, sem_out)

  for bl in range(B // NC):
    b = core * (B // NC) + bl
    phase_init(b)
    phase_reduce(b, is_init=True, last_f=False)
    def upd_body(it, carry):
      phase_update(b)
      phase_reduce(b, is_init=False, last_f=(it == NUM_UPDATES - 1))
      return carry
    lax.fori_loop(0, NUM_UPDATES, upd_body, 0)
    phase_final(b)


@jax.jit
def _ssn(img_lab, init_spIndx):
  mesh = plsc.VectorSubcoreMesh(core_axis_name="c", subcore_axis_name="s")
  f = pl.kernel(
      _ssn_body,
      out_type=[
          jax.ShapeDtypeStruct((B, 5, H, W), jnp.float32),
          jax.ShapeDtypeStruct((B, 5, K), jnp.float32),
          jax.ShapeDtypeStruct((B, 9, H, W), jnp.float32),
          jax.ShapeDtypeStruct((B, 1, H, W), jnp.int32),
      ],
      mesh=mesh,
      compiler_params=pltpu.CompilerParams(needs_layout_passes=False),
      scratch_types=[
          pltpu.VMEM((L * ACC_W,), jnp.float32),
          pltpu.VMEM((ACC_W,), jnp.float32),
          pltpu.VMEM((ACC_W,), jnp.float32),
          pltpu.VMEM((2 * 3 * W,), jnp.float32),
          pltpu.VMEM((2 * W,), jnp.int32),
          pltpu.VMEM((CH_W,), jnp.float32),
          pltpu.VMEM((5 * SEG_PER_TILE,), jnp.float32),
          pltpu.VMEM((9 * W,), jnp.float32),
          pltpu.VMEM((W,), jnp.int32),
          pltpu.VMEM((5 * W,), jnp.float32),
          pltpu.VMEM_SHARED((NS * ACC_W,), jnp.float32),
          pltpu.VMEM_SHARED((ACC_W,), jnp.float32),
          pltpu.SemaphoreType.DMA,
          pltpu.SemaphoreType.DMA,
          pltpu.SemaphoreType.DMA,
      ],
  )
  return f(img_lab, init_spIndx.astype(jnp.int32))


def kernel(img_lab, init_spIndx):
  pFeat, spFeat, psp_assoc, final_spIndx = _ssn(img_lab, init_spIndx)
  return (pFeat, spFeat, psp_assoc, final_spIndx)


# speedup vs baseline: 1.0253x; 1.0182x over previous
import functools

import jax
import jax.numpy as jnp
from jax import lax
from jax.experimental import pallas as pl
from jax.experimental.pallas import tpu as pltpu
from jax.experimental.pallas import tpu_sc as plsc

B, H, W = 4, 512, 512
KH, KW = 32, 32
K = KH * KW
YX_SCALE = KH / (0.4 * H)
LAB_SCALE = 0.26
NUM_UPDATES = 4

NC, NS, L = 2, 16, 16
ROWS_PER_TILE = H // NS
VECS_PER_ROW = W // L
ST = 6
ACC_W = K * ST
SEG_PER_TILE = K // NS
CH_W = SEG_PER_TILE * ST


def _ssn_body(img_ref, sidx_ref, pfeat_ref, spfeat_ref, assoc_ref, fidx_ref,
              acc, table, red, labbuf, idxbuf, tstage, spout,
              assocbuf, fidxbuf, pfeatbuf, slots, shtab, sem_in, sem_in2,
              sem_out):
  core = lax.axis_index("c")
  tile = lax.axis_index("s")
  iota = lax.iota(jnp.int32, 16)
  lane_base = iota * ACC_W
  zeros16 = jnp.zeros((16,), jnp.float32)
  ones16 = jnp.ones((16,), jnp.float32)

  def _tree_sum(vals):
    while len(vals) > 1:
      vals = [a + b for a, b in zip(vals[::2], vals[1::2])] + (
          [vals[-1]] if len(vals) % 2 else [])
    return vals[0]

  def _tree_max(vals):
    while len(vals) > 1:
      vals = [jnp.maximum(a, b) for a, b in zip(vals[::2], vals[1::2])] + (
          [vals[-1]] if len(vals) % 2 else [])
    return vals[0]

  @plsc.parallel_loop(0, (L * ACC_W) // (16 * 8), unroll=4)
  def _zero_body(i):
    for u in range(8):
      acc[pl.ds((i * 8 + u) * 16, 16)] = zeros16

  def _fire_row(b, r, par, sem):
    pltpu.async_copy(sidx_ref.at[b, 0, r], idxbuf.at[pl.ds(par * W, W)], sem)
    for c in range(3):
      pltpu.async_copy(img_ref.at[b, c, r],
                       labbuf.at[pl.ds((par * 3 + c) * W, W)], sem)

  def _drain_row(b, r, par, sem):
    pltpu.make_async_copy(sidx_ref.at[b, 0, r],
                          idxbuf.at[pl.ds(par * W, W)], sem).wait()
    for c in range(3):
      pltpu.make_async_copy(img_ref.at[b, c, r],
                            labbuf.at[pl.ds((par * 3 + c) * W, W)], sem).wait()

  def run_rows(b, vec_loop):
    _fire_row(b, tile * ROWS_PER_TILE, 0, sem_in)
    def row_body(ri, carry):
      r = tile * ROWS_PER_TILE + ri
      even = (ri & 1) == 0
      last = ri >= ROWS_PER_TILE - 1

      @pl.when(jnp.logical_and(jnp.logical_not(last), even))
      def _():
        _fire_row(b, r + 1, 1, sem_in2)

      @pl.when(jnp.logical_and(jnp.logical_not(last), jnp.logical_not(even)))
      def _():
        _fire_row(b, r + 1, 0, sem_in)

      @pl.when(even)
      def _():
        _drain_row(b, r, 0, sem_in)

      @pl.when(jnp.logical_not(even))
      def _():
        _drain_row(b, r, 1, sem_in2)

      vec_loop(ri, r, ri & 1)
      return carry
    lax.fori_loop(0, ROWS_PER_TILE, row_body, 0)

  def pixel_feats(r, v, par):
    y16 = lax.broadcast_in_dim(r, (16,), ())
    p0 = y16.astype(jnp.float32) * YX_SCALE
    x16 = iota + v * 16
    p1 = x16.astype(jnp.float32) * YX_SCALE
    lab = [labbuf[pl.ds((par * 3 + c) * W + v * 16, 16)] * LAB_SCALE
           for c in range(3)]
    return [p0, p1] + lab

  def cand_parts(s_i):
    sh = s_i >> 5
    sw = s_i & 31
    shm = jnp.maximum(sh - 1, 0)
    shp = jnp.minimum(sh + 1, KH - 1)
    swm = jnp.maximum(sw - 1, 0)
    swp = jnp.minimum(sw + 1, KW - 1)
    rh6 = [shm * (KW * ST), sh * (KW * ST), shp * (KW * ST)]
    cw6 = [swm * ST, sw * ST, swp * ST]
    return rh6, cw6

  def softmax9_dot(p, k6s):
    lams = []
    for k6 in k6s:
      ts = [plsc.load_gather(table, [k6 + c if c else k6]) for c in range(6)]
      prods = [p[c] * ts[c] for c in range(5)]
      lam = ((prods[0] + prods[1]) + (prods[2] + prods[3])) + (prods[4] - ts[5])
      lams.append(lam)
    m = _tree_max(lams)
    es = [jnp.exp(lam - m) for lam in lams]
    inv = 1.0 / _tree_sum(es)
    return es, inv

  def softmax9(p, k6s):
    lams = []
    for k6 in k6s:
      sqs = []
      for c in range(5):
        df = p[c] - plsc.load_gather(table, [k6 + c if c else k6])
        sqs.append(df * df)
      lams.append(-(((sqs[0] + sqs[1]) + (sqs[2] + sqs[3])) + sqs[4]))
    m = _tree_max(lams)
    es = [jnp.exp(lam - m) for lam in lams]
    inv = 1.0 / _tree_sum(es)
    return es, inv

  def phase_init(b):
    def vec_loop(ri, r, par):
      @plsc.parallel_loop(0, VECS_PER_ROW, unroll=2)
      def vec_body(v):
        s_i = idxbuf[pl.ds(par * W + v * 16, 16)]
        p = pixel_feats(r, v, par)
        base = s_i * ST + lane_base
        for c in range(5):
          plsc.addupdate_scatter(acc, [base + c if c else base], p[c])
        plsc.addupdate_scatter(acc, [base + 5], ones16)
    run_rows(b, vec_loop)

  def phase_update(b):
    def vec_loop(ri, r, par):
      def vec_body(v, vc):
        s_i = idxbuf[pl.ds(par * W + v * 16, 16)]
        p = pixel_feats(r, v, par)
        rh6, cw6 = cand_parts(s_i)
        k6s = [rh6[a] + cw6[bb] for a in range(3) for bb in range(3)]
        es, inv = softmax9_dot(p, k6s)
        for j in range(9):
          a = es[j] * inv
          base = (rh6[j // 3] + cw6[j % 3]) + lane_base
          for c in range(5):
            plsc.addupdate_scatter(acc, [base + c if c else base], a * p[c])
          plsc.addupdate_scatter(acc, [base + 5], a)
        return vc
      lax.fori_loop(0, VECS_PER_ROW, vec_body, 0)
    run_rows(b, vec_loop)

  def phase_reduce(b, is_init, last_f):
    @plsc.parallel_loop(0, ACC_W // 16, unroll=2)
    def red_body(i):
      base = i * 16
      vals = [acc[pl.ds(l * ACC_W + base, 16)] for l in range(L)]
      red[pl.ds(base, 16)] = _tree_sum(vals)
      for l in range(L):
        acc[pl.ds(l * ACC_W + base, 16)] = zeros16
    pltpu.sync_copy(red, slots.at[pl.ds(tile * ACC_W, ACC_W)])
    plsc.subcore_barrier()
    cps = [pltpu.async_copy(slots.at[pl.ds(i * ACC_W + tile * CH_W, CH_W)],
                            red.at[pl.ds(i * CH_W, CH_W)], sem_in)
           for i in range(NS)]
    for cp in cps:
      cp.wait()
    @plsc.parallel_loop(0, CH_W // 16, unroll=2)
    def sum_body(i):
      base = i * 16
      vals = [red[pl.ds(l * CH_W + base, 16)] for l in range(NS)]
      tstage[pl.ds(base, 16)] = _tree_sum(vals)
    for vv in range(SEG_PER_TILE // 16):
      kloc6 = (iota + vv * 16) * ST
      num = [plsc.load_gather(tstage, [kloc6 + c if c else kloc6])
             for c in range(5)]
      den = plsc.load_gather(tstage, [kloc6 + 5])
      if is_init:
        dadj = jnp.maximum(den, 1e-12)
      else:
        dadj = den + 1e-10
      invd = 1.0 / dadj
      fcs = [num[c] * invd for c in range(5)]
      sqs = [fc * fc for fc in fcs]
      nrm = ((sqs[0] + sqs[1]) + (sqs[2] + sqs[3])) + sqs[4]
      for c in range(5):
        val = jnp.where(last_f, fcs[c], fcs[c] * 2.0)
        plsc.store_scatter(tstage, [kloc6 + c if c else kloc6], val)
        if not is_init:
          spout[pl.ds(c * SEG_PER_TILE + vv * 16, 16)] = fcs[c]
      plsc.store_scatter(tstage, [kloc6 + 5], nrm)
    pltpu.sync_copy(tstage, shtab.at[pl.ds(tile * CH_W, CH_W)])
    if not is_init:
      for c in range(5):
        pltpu.sync_copy(spout.at[pl.ds(c * SEG_PER_TILE, SEG_PER_TILE)],
                        spfeat_ref.at[b, c, pl.ds(tile * SEG_PER_TILE,
                                                  SEG_PER_TILE)])
    plsc.subcore_barrier()
    pltpu.sync_copy(shtab, table)

  def phase_final(b):
    def vec_loop(ri, r, par):
      @pl.when(ri > 0)
      def _drain():
        _fire_outputs(b, r - 1, wait_only=True)

      def vec_body(v, vc):
        s_i = idxbuf[pl.ds(par * W + v * 16, 16)]
        p = pixel_feats(r, v, par)
        sh = s_i >> 5
        sw = s_i & 31
        shm = jnp.maximum(sh - 1, 0)
        shp = jnp.minimum(sh + 1, KH - 1)
        swm = jnp.maximum(sw - 1, 0)
        swp = jnp.minimum(sw + 1, KW - 1)
        rh = [shm * KW, sh * KW, shp * KW]
        cw = [swm, sw, swp]
        ks = [rh[a] + cw[bb] for a in range(3) for bb in range(3)]
        k6s = [kk * ST for kk in ks]
        es, inv = softmax9(p, k6s)
        cands = []
        for j in range(9):
          a = es[j] * inv
          assocbuf[pl.ds(j * W + v * 16, 16)] = a
          cands.append((a, ks[j]))
        while len(cands) > 1:
          nxt = []
          for (av, ak), (bv, bk2) in zip(cands[::2], cands[1::2]):
            gt = bv > av
            nxt.append((jnp.where(gt, bv, av), jnp.where(gt, bk2, ak)))
          if len(cands) % 2:
            nxt.append(cands[-1])
          cands = nxt
        fidxbuf[pl.ds(v * 16, 16)] = cands[0][1]
        for c in range(5):
          pfeatbuf[pl.ds(c * W + v * 16, 16)] = p[c]
        return vc
      lax.fori_loop(0, VECS_PER_ROW, vec_body, 0)
      _fire_outputs(b, r, wait_only=False)
    run_rows(b, vec_loop)
    _fire_outputs(b, tile * ROWS_PER_TILE + ROWS_PER_TILE - 1, wait_only=True)

  def _fire_outputs(b, r, wait_only):
    descs = []
    for j in range(9):
      descs.append((assocbuf.at[pl.ds(j * W, W)], assoc_ref.at[b, j, r]))
    descs.append((fidxbuf, fidx_ref.at[b, 0, r]))
    for c in range(5):
      descs.append((pfeatbuf.at[pl.ds(c * W, W)], pfeat_ref.at[b, c, r]))
    if wait_only:
      for src, dst in descs:
        pltpu.make_async_copy(src, dst, sem_out).wait()
    else:
      for src, dst in descs:
        pltpu.async_copy(src, dst, sem_out)

  for bl in range(B // NC):
    b = core * (B // NC) + bl
    phase_init(b)
    phase_reduce(b, is_init=True, last_f=False)
    def upd_body(it, carry):
      phase_update(b)
      phase_reduce(b, is_init=False, last_f=(it == NUM_UPDATES - 1))
      return carry
    lax.fori_loop(0, NUM_UPDATES, upd_body, 0)
    phase_final(b)


@jax.jit
def _ssn(img_lab, init_spIndx):
  mesh = plsc.VectorSubcoreMesh(core_axis_name="c", subcore_axis_name="s")
  f = pl.kernel(
      _ssn_body,
      out_type=[
          jax.ShapeDtypeStruct((B, 5, H, W), jnp.float32),
          jax.ShapeDtypeStruct((B, 5, K), jnp.float32),
          jax.ShapeDtypeStruct((B, 9, H, W), jnp.float32),
          jax.ShapeDtypeStruct((B, 1, H, W), jnp.int32),
      ],
      mesh=mesh,
      compiler_params=pltpu.CompilerParams(needs_layout_passes=False),
      scratch_types=[
          pltpu.VMEM((L * ACC_W,), jnp.float32),
          pltpu.VMEM((ACC_W,), jnp.float32),
          pltpu.VMEM((ACC_W,), jnp.float32),
          pltpu.VMEM((2 * 3 * W,), jnp.float32),
          pltpu.VMEM((2 * W,), jnp.int32),
          pltpu.VMEM((CH_W,), jnp.float32),
          pltpu.VMEM((5 * SEG_PER_TILE,), jnp.float32),
          pltpu.VMEM((9 * W,), jnp.float32),
          pltpu.VMEM((W,), jnp.int32),
          pltpu.VMEM((5 * W,), jnp.float32),
          pltpu.VMEM_SHARED((NS * ACC_W,), jnp.float32),
          pltpu.VMEM_SHARED((ACC_W,), jnp.float32),
          pltpu.SemaphoreType.DMA,
          pltpu.SemaphoreType.DMA,
          pltpu.SemaphoreType.DMA,
      ],
  )
  return f(img_lab, init_spIndx.astype(jnp.int32))


def kernel(img_lab, init_spIndx):
  pFeat, spFeat, psp_assoc, final_spIndx = _ssn(img_lab, init_spIndx)
  return (pFeat, spFeat, psp_assoc, final_spIndx)
